# Initial kernel scaffold; baseline (speedup 1.0000x reference)
import jax
import jax.numpy as jnp
from jax.experimental import pallas as pl


def kernel(x, x0, edge_index, W1, W2):
    raise NotImplementedError("write your pallas kernel here")


# speedup vs baseline: 11.4056x; 11.4056x over previous
import functools

import jax
import jax.numpy as jnp
from jax import lax
from jax.experimental import pallas as pl
from jax.experimental.pallas import tpu as pltpu
from jax.experimental.pallas import tpu_sc as plsc

N = 10000
E = 160000
D = 256
H = D // 2
NPAD = 10240
ROWS_PER_TILE = 640
E_PER_TILE = E // 16
CHUNK = 128
NFULL = E_PER_TILE // CHUNK
REM = E_PER_TILE - NFULL * CHUNK

_MESH = plsc.VectorSubcoreMesh(core_axis_name="c", subcore_axis_name="s")


def _fill(ref, n, value):
    vec = jnp.full((16,), value, dtype=ref.dtype)

    def body(i, _):
        ref[pl.ds(i * 16, 16)] = vec
        return 0

    lax.fori_loop(0, n // 16, body, 0, unroll=False)


@functools.partial(
    pl.kernel,
    out_type=jax.ShapeDtypeStruct((NPAD,), jnp.float32),
    mesh=_MESH,
    scratch_types=[
        pltpu.VMEM((CHUNK,), jnp.int32),
        pltpu.VMEM((REM,), jnp.int32),
        pltpu.VMEM((CHUNK,), jnp.float32),
        pltpu.VMEM((REM,), jnp.float32),
        pltpu.VMEM((ROWS_PER_TILE,), jnp.float32),
        pltpu.VMEM_SHARED((NPAD,), jnp.float32),
    ],
)
def _degree_kernel(dst_hbm, deg_hbm, idx_v, idx_r, ones_v, ones_r, zeros_v,
                   deg_sh):
    c = lax.axis_index("c")
    s = lax.axis_index("s")

    @pl.when(c == 0)
    def _():
        _fill(zeros_v, ROWS_PER_TILE, 0.0)
        _fill(ones_v, CHUNK, 1.0)
        _fill(ones_r, REM, 1.0)
        pltpu.sync_copy(zeros_v, deg_sh.at[pl.ds(s * ROWS_PER_TILE,
                                                 ROWS_PER_TILE)])

    plsc.subcore_barrier()

    @pl.when(c == 0)
    def _():
        base = s * E_PER_TILE

        def chunk(i, _):
            pltpu.sync_copy(dst_hbm.at[pl.ds(base + i * CHUNK, CHUNK)], idx_v)
            pltpu.sync_copy(ones_v, deg_sh.at[idx_v], add=True)
            return 0

        lax.fori_loop(0, NFULL, chunk, 0, unroll=False)
        pltpu.sync_copy(dst_hbm.at[pl.ds(base + NFULL * CHUNK, REM)], idx_r)
        pltpu.sync_copy(ones_r, deg_sh.at[idx_r], add=True)

    plsc.subcore_barrier()

    @pl.when(c == 0)
    def _():
        sl = pl.ds(s * ROWS_PER_TILE, ROWS_PER_TILE)
        pltpu.sync_copy(deg_sh.at[sl], deg_hbm.at[sl])


@functools.partial(
    pl.kernel,
    out_type=(
        jax.ShapeDtypeStruct((N, H), jnp.float32),
        jax.ShapeDtypeStruct((N, H), jnp.float32),
    ),
    mesh=_MESH,
    scratch_types=[
        pltpu.VMEM((CHUNK,), jnp.int32),
        pltpu.VMEM((CHUNK,), jnp.int32),
        pltpu.VMEM((REM,), jnp.int32),
        pltpu.VMEM((REM,), jnp.int32),
        pltpu.VMEM((CHUNK, H), jnp.float32),
        pltpu.VMEM((REM, H), jnp.float32),
        pltpu.VMEM_SHARED((NPAD, H), jnp.float32),
        pltpu.SemaphoreType.DMA,
    ],
)
def _edge_kernel(zlo_hbm, zhi_hbm, src_hbm, dst_hbm, outlo_hbm, outhi_hbm,
                 src_v, dst_v, src_r, dst_r, rows_v, rows_r, acc_sh, sem):
    c = lax.axis_index("c")
    s = lax.axis_index("s")

    last = N - 15 * ROWS_PER_TILE

    def init(z_ref):
        @pl.when(s < 15)
        def _():
            sl = pl.ds(s * ROWS_PER_TILE, ROWS_PER_TILE)
            pltpu.sync_copy(z_ref.at[sl], acc_sh.at[sl])

        @pl.when(s == 15)
        def _():
            sl = pl.ds(15 * ROWS_PER_TILE, last)
            pltpu.sync_copy(z_ref.at[sl], acc_sh.at[sl])

    def edges(z_ref):
        base = s * E_PER_TILE

        def chunk(i, _):
            b = base + i * CHUNK
            pltpu.sync_copy(src_hbm.at[pl.ds(b, CHUNK)], src_v)
            pltpu.sync_copy(dst_hbm.at[pl.ds(b, CHUNK)], dst_v)
            pltpu.async_copy(z_ref.at[src_v], rows_v, sem).wait()
            pltpu.sync_copy(rows_v, acc_sh.at[dst_v], add=True)
            return 0

        lax.fori_loop(0, NFULL, chunk, 0, unroll=False)
        b = base + NFULL * CHUNK
        pltpu.sync_copy(src_hbm.at[pl.ds(b, REM)], src_r)
        pltpu.sync_copy(dst_hbm.at[pl.ds(b, REM)], dst_r)
        pltpu.async_copy(z_ref.at[src_r], rows_r, sem).wait()
        pltpu.sync_copy(rows_r, acc_sh.at[dst_r], add=True)

    def writeout(out_ref):
        @pl.when(s < 15)
        def _():
            sl = pl.ds(s * ROWS_PER_TILE, ROWS_PER_TILE)
            pltpu.sync_copy(acc_sh.at[sl], out_ref.at[sl])

        @pl.when(s == 15)
        def _():
            sl = pl.ds(15 * ROWS_PER_TILE, last)
            pltpu.sync_copy(acc_sh.at[sl], out_ref.at[sl])

    @pl.when(c == 0)
    def _():
        init(zlo_hbm)

    @pl.when(c == 1)
    def _():
        init(zhi_hbm)

    plsc.subcore_barrier()

    @pl.when(c == 0)
    def _():
        edges(zlo_hbm)

    @pl.when(c == 1)
    def _():
        edges(zhi_hbm)

    plsc.subcore_barrier()

    @pl.when(c == 0)
    def _():
        writeout(outlo_hbm)

    @pl.when(c == 1)
    def _():
        writeout(outhi_hbm)


RBLK = 400


def _project_body(x_ref, x0_ref, w1_ref, w2_ref, deg_ref, zlo_ref, zhi_ref,
                  base_ref, dinv_ref):
    deg = deg_ref[...] + 1.0
    dinv = lax.rsqrt(deg)
    y = lax.dot_general(x_ref[...], w1_ref[...], (((1,), (1,)), ((), ())),
                        preferred_element_type=jnp.float32)
    z = (0.9 * dinv) * y
    zlo_ref[...] = z[:, :H]
    zhi_ref[...] = z[:, H:]
    base_ref[...] = 0.1 * lax.dot_general(
        x0_ref[...], w2_ref[...], (((1,), (1,)), ((), ())),
        preferred_element_type=jnp.float32)
    dinv_ref[...] = dinv


def _finish_body(acclo_ref, acchi_ref, base_ref, dinv_ref, out_ref):
    acc = jnp.concatenate([acclo_ref[...], acchi_ref[...]], axis=1)
    out_ref[...] = base_ref[...] + dinv_ref[...] * acc


def _project(x, x0, W1, W2, deg2d):
    grid = (N // RBLK,)
    return pl.pallas_call(
        _project_body,
        grid=grid,
        in_specs=[
            pl.BlockSpec((RBLK, D), lambda i: (i, 0)),
            pl.BlockSpec((RBLK, D), lambda i: (i, 0)),
            pl.BlockSpec((D, D), lambda i: (0, 0)),
            pl.BlockSpec((D, D), lambda i: (0, 0)),
            pl.BlockSpec((RBLK, 1), lambda i: (i, 0)),
        ],
        out_specs=[
            pl.BlockSpec((RBLK, H), lambda i: (i, 0)),
            pl.BlockSpec((RBLK, H), lambda i: (i, 0)),
            pl.BlockSpec((RBLK, D), lambda i: (i, 0)),
            pl.BlockSpec((RBLK, 1), lambda i: (i, 0)),
        ],
        out_shape=[
            jax.ShapeDtypeStruct((N, H), jnp.float32),
            jax.ShapeDtypeStruct((N, H), jnp.float32),
            jax.ShapeDtypeStruct((N, D), jnp.float32),
            jax.ShapeDtypeStruct((N, 1), jnp.float32),
        ],
    )(x, x0, W1, W2, deg2d)


def _finish(acc_lo, acc_hi, base, dinv2d):
    grid = (N // RBLK,)
    return pl.pallas_call(
        _finish_body,
        grid=grid,
        in_specs=[
            pl.BlockSpec((RBLK, H), lambda i: (i, 0)),
            pl.BlockSpec((RBLK, H), lambda i: (i, 0)),
            pl.BlockSpec((RBLK, D), lambda i: (i, 0)),
            pl.BlockSpec((RBLK, 1), lambda i: (i, 0)),
        ],
        out_specs=pl.BlockSpec((RBLK, D), lambda i: (i, 0)),
        out_shape=jax.ShapeDtypeStruct((N, D), jnp.float32),
    )(acc_lo, acc_hi, base, dinv2d)


def kernel(x, x0, edge_index, W1, W2):
    src = edge_index[0].astype(jnp.int32)
    dst = edge_index[1].astype(jnp.int32)

    deg = _degree_kernel(dst)
    deg2d = deg[:N].reshape(N, 1)

    z_lo, z_hi, base, dinv2d = _project(x, x0, W1, W2, deg2d)
    acc_lo, acc_hi = _edge_kernel(z_lo, z_hi, src, dst)
    return _finish(acc_lo, acc_hi, base, dinv2d)


# speedup vs baseline: 16.0645x; 1.4085x over previous
import functools

import jax
import jax.numpy as jnp
from jax import lax
from jax.experimental import pallas as pl
from jax.experimental.pallas import tpu as pltpu
from jax.experimental.pallas import tpu_sc as plsc

N = 10000
E = 160000
D = 256
H = D // 2
NPAD = 10240
ROWS_PER_TILE = 640
CHUNK = 128
ECHUNKS = E // CHUNK
EPAIRS = (ECHUNKS // 16) // 2
DCHUNKS = (E // 2) // CHUNK

_MESH = plsc.VectorSubcoreMesh(core_axis_name="c", subcore_axis_name="s")


def _fill(ref, n, value):
    vec = jnp.full((16,), value, dtype=ref.dtype)

    def body(i, _):
        ref[pl.ds(i * 16, 16)] = vec
        return 0

    lax.fori_loop(0, n // 16, body, 0, unroll=False)


@functools.partial(
    pl.kernel,
    out_type=jax.ShapeDtypeStruct((2, NPAD), jnp.float32),
    mesh=_MESH,
    scratch_types=[
        pltpu.VMEM((CHUNK,), jnp.int32),
        pltpu.VMEM((CHUNK,), jnp.float32),
        pltpu.VMEM((ROWS_PER_TILE,), jnp.float32),
        pltpu.VMEM_SHARED((NPAD,), jnp.float32),
    ],
)
def _degree_kernel(dst_hbm, deg_hbm, idx_v, ones_v, zeros_v, deg_sh):
    c = lax.axis_index("c")
    s = lax.axis_index("s")

    _fill(zeros_v, ROWS_PER_TILE, 0.0)
    _fill(ones_v, CHUNK, 1.0)
    pltpu.sync_copy(zeros_v, deg_sh.at[pl.ds(s * ROWS_PER_TILE,
                                             ROWS_PER_TILE)])

    plsc.subcore_barrier()

    base = c * (E // 2)

    def chunk(g):
        pltpu.sync_copy(dst_hbm.at[pl.ds(base + g * CHUNK, CHUNK)], idx_v)
        pltpu.sync_copy(ones_v, deg_sh.at[idx_v], add=True)

    def body(k, _):
        chunk(s + 16 * k)
        return 0

    lax.fori_loop(0, DCHUNKS // 16, body, 0, unroll=False)

    @pl.when(s == 0)
    def _():
        chunk(DCHUNKS - 1)

    plsc.subcore_barrier()

    sl = pl.ds(s * ROWS_PER_TILE, ROWS_PER_TILE)
    pltpu.sync_copy(deg_sh.at[sl], deg_hbm.at[c, sl])


@functools.partial(
    pl.kernel,
    out_type=(
        jax.ShapeDtypeStruct((N, H), jnp.float32),
        jax.ShapeDtypeStruct((N, H), jnp.float32),
    ),
    mesh=_MESH,
    scratch_types=[
        pltpu.VMEM((CHUNK,), jnp.int32),
        pltpu.VMEM((CHUNK,), jnp.int32),
        pltpu.VMEM((CHUNK,), jnp.int32),
        pltpu.VMEM((CHUNK,), jnp.int32),
        pltpu.VMEM((CHUNK, H), jnp.float32),
        pltpu.VMEM((CHUNK, H), jnp.float32),
        pltpu.VMEM_SHARED((NPAD, H), jnp.float32),
        pltpu.SemaphoreType.DMA,
        pltpu.SemaphoreType.DMA,
    ],
)
def _edge_kernel(zlo_hbm, zhi_hbm, src_hbm, dst_hbm, outlo_hbm, outhi_hbm,
                 src_v0, dst_v0, src_v1, dst_v1,
                 rows_v0, rows_v1, acc_sh, sem0, sem1):
    c = lax.axis_index("c")
    s = lax.axis_index("s")

    last = N - 15 * ROWS_PER_TILE

    def init(z_ref):
        @pl.when(s < 15)
        def _():
            sl = pl.ds(s * ROWS_PER_TILE, ROWS_PER_TILE)
            pltpu.sync_copy(z_ref.at[sl], acc_sh.at[sl])

        @pl.when(s == 15)
        def _():
            sl = pl.ds(15 * ROWS_PER_TILE, last)
            pltpu.sync_copy(z_ref.at[sl], acc_sh.at[sl])

    def edges(z_ref):
        def load_idx(g, s_v, d_v):
            b = g * CHUNK
            pltpu.sync_copy(src_hbm.at[pl.ds(b, CHUNK)], s_v)
            pltpu.sync_copy(dst_hbm.at[pl.ds(b, CHUNK)], d_v)

        load_idx(s, src_v0, dst_v0)
        pltpu.async_copy(z_ref.at[src_v0], rows_v0, sem0)

        def pair(p, _):
            load_idx(s + 16 * (2 * p + 1), src_v1, dst_v1)
            pltpu.async_copy(z_ref.at[src_v1], rows_v1, sem1)
            pltpu.make_async_copy(z_ref.at[src_v0], rows_v0, sem0).wait()
            pltpu.sync_copy(rows_v0, acc_sh.at[dst_v0], add=True)

            @pl.when(p + 1 < EPAIRS)
            def _():
                load_idx(s + 16 * (2 * p + 2), src_v0, dst_v0)
                pltpu.async_copy(z_ref.at[src_v0], rows_v0, sem0)

            pltpu.make_async_copy(z_ref.at[src_v1], rows_v1, sem1).wait()
            pltpu.sync_copy(rows_v1, acc_sh.at[dst_v1], add=True)
            return 0

        lax.fori_loop(0, EPAIRS, pair, 0, unroll=False)

        @pl.when(s < ECHUNKS - 16 * 2 * EPAIRS)
        def _():
            g = 16 * 2 * EPAIRS + s
            load_idx(g, src_v0, dst_v0)
            pltpu.async_copy(z_ref.at[src_v0], rows_v0, sem0).wait()
            pltpu.sync_copy(rows_v0, acc_sh.at[dst_v0], add=True)

    def writeout(out_ref):
        @pl.when(s < 15)
        def _():
            sl = pl.ds(s * ROWS_PER_TILE, ROWS_PER_TILE)
            pltpu.sync_copy(acc_sh.at[sl], out_ref.at[sl])

        @pl.when(s == 15)
        def _():
            sl = pl.ds(15 * ROWS_PER_TILE, last)
            pltpu.sync_copy(acc_sh.at[sl], out_ref.at[sl])

    @pl.when(c == 0)
    def _():
        init(zlo_hbm)

    @pl.when(c == 1)
    def _():
        init(zhi_hbm)

    plsc.subcore_barrier()

    @pl.when(c == 0)
    def _():
        edges(zlo_hbm)

    @pl.when(c == 1)
    def _():
        edges(zhi_hbm)

    plsc.subcore_barrier()

    @pl.when(c == 0)
    def _():
        writeout(outlo_hbm)

    @pl.when(c == 1)
    def _():
        writeout(outhi_hbm)


RBLK = 400


def _project_body(x_ref, x0_ref, w1_ref, w2_ref, deg0_ref, deg1_ref, zlo_ref,
                  zhi_ref, base_ref, dinv_ref):
    deg = deg0_ref[...] + deg1_ref[...] + 1.0
    dinv = lax.rsqrt(deg)
    y = lax.dot_general(x_ref[...], w1_ref[...], (((1,), (1,)), ((), ())),
                        preferred_element_type=jnp.float32,
                        precision=lax.Precision.HIGHEST)
    z = (0.9 * dinv) * y
    zlo_ref[...] = z[:, :H]
    zhi_ref[...] = z[:, H:]
    base_ref[...] = 0.1 * lax.dot_general(
        x0_ref[...], w2_ref[...], (((1,), (1,)), ((), ())),
        preferred_element_type=jnp.float32,
        precision=lax.Precision.HIGHEST)
    dinv_ref[...] = dinv


def _finish_body(acclo_ref, acchi_ref, base_ref, dinv_ref, out_ref):
    acc = jnp.concatenate([acclo_ref[...], acchi_ref[...]], axis=1)
    out_ref[...] = base_ref[...] + dinv_ref[...] * acc


def _project(x, x0, W1, W2, deg2d0, deg2d1):
    grid = (N // RBLK,)
    return pl.pallas_call(
        _project_body,
        grid=grid,
        in_specs=[
            pl.BlockSpec((RBLK, D), lambda i: (i, 0)),
            pl.BlockSpec((RBLK, D), lambda i: (i, 0)),
            pl.BlockSpec((D, D), lambda i: (0, 0)),
            pl.BlockSpec((D, D), lambda i: (0, 0)),
            pl.BlockSpec((RBLK, 1), lambda i: (i, 0)),
            pl.BlockSpec((RBLK, 1), lambda i: (i, 0)),
        ],
        out_specs=[
            pl.BlockSpec((RBLK, H), lambda i: (i, 0)),
            pl.BlockSpec((RBLK, H), lambda i: (i, 0)),
            pl.BlockSpec((RBLK, D), lambda i: (i, 0)),
            pl.BlockSpec((RBLK, 1), lambda i: (i, 0)),
        ],
        out_shape=[
            jax.ShapeDtypeStruct((N, H), jnp.float32),
            jax.ShapeDtypeStruct((N, H), jnp.float32),
            jax.ShapeDtypeStruct((N, D), jnp.float32),
            jax.ShapeDtypeStruct((N, 1), jnp.float32),
        ],
    )(x, x0, W1, W2, deg2d0, deg2d1)


def _finish(acc_lo, acc_hi, base, dinv2d):
    grid = (N // RBLK,)
    return pl.pallas_call(
        _finish_body,
        grid=grid,
        in_specs=[
            pl.BlockSpec((RBLK, H), lambda i: (i, 0)),
            pl.BlockSpec((RBLK, H), lambda i: (i, 0)),
            pl.BlockSpec((RBLK, D), lambda i: (i, 0)),
            pl.BlockSpec((RBLK, 1), lambda i: (i, 0)),
        ],
        out_specs=pl.BlockSpec((RBLK, D), lambda i: (i, 0)),
        out_shape=jax.ShapeDtypeStruct((N, D), jnp.float32),
    )(acc_lo, acc_hi, base, dinv2d)


def kernel(x, x0, edge_index, W1, W2):
    src = edge_index[0].astype(jnp.int32)
    dst = edge_index[1].astype(jnp.int32)

    degp = _degree_kernel(dst)
    deg2d0 = degp[0, :N].reshape(N, 1)
    deg2d1 = degp[1, :N].reshape(N, 1)

    z_lo, z_hi, base, dinv2d = _project(x, x0, W1, W2, deg2d0, deg2d1)
    acc_lo, acc_hi = _edge_kernel(z_lo, z_hi, src, dst)
    return _finish(acc_lo, acc_hi, base, dinv2d)


# speedup vs baseline: 20.4113x; 1.2706x over previous
import functools

import jax
import jax.numpy as jnp
from jax import lax
from jax.experimental import pallas as pl
from jax.experimental.pallas import tpu as pltpu
from jax.experimental.pallas import tpu_sc as plsc

N = 10000
E = 160000
D = 256
H = D // 2
NPAD = 10240
ROWS_PER_TILE = 640
CHUNK = 128
ECHUNKS = E // CHUNK
EPAIRS = (ECHUNKS // 16) // 2
DCHUNKS = (E // 2) // CHUNK

_MESH = plsc.VectorSubcoreMesh(core_axis_name="c", subcore_axis_name="s")


def _fill(ref, n, value):
    vec = jnp.full((16,), value, dtype=ref.dtype)

    def body(i, _):
        ref[pl.ds(i * 16, 16)] = vec
        return 0

    lax.fori_loop(0, n // 16, body, 0, unroll=False)


@functools.partial(
    pl.kernel,
    out_type=jax.ShapeDtypeStruct((2, NPAD), jnp.float32),
    mesh=_MESH,
    scratch_types=[
        pltpu.VMEM((CHUNK,), jnp.int32),
        pltpu.VMEM((CHUNK,), jnp.int32),
        pltpu.VMEM((CHUNK,), jnp.int32),
        pltpu.VMEM((CHUNK,), jnp.float32),
        pltpu.VMEM((ROWS_PER_TILE,), jnp.float32),
        pltpu.VMEM_SHARED((NPAD,), jnp.float32),
        pltpu.SemaphoreType.DMA,
        pltpu.SemaphoreType.DMA,
        pltpu.SemaphoreType.DMA,
    ],
)
def _degree_kernel(dst_hbm, deg_hbm, idx_a, idx_b, idx_c, ones_v, zeros_v,
                   deg_sh, sem_a, sem_b, sem_c):
    c = lax.axis_index("c")
    s = lax.axis_index("s")

    _fill(zeros_v, ROWS_PER_TILE, 0.0)
    _fill(ones_v, CHUNK, 1.0)
    pltpu.sync_copy(zeros_v, deg_sh.at[pl.ds(s * ROWS_PER_TILE,
                                             ROWS_PER_TILE)])

    plsc.subcore_barrier()

    base = c * (E // 2)
    nk = DCHUNKS // 16

    def src_at(k):
        return dst_hbm.at[pl.ds(base + (s + 16 * k) * CHUNK, CHUNK)]

    def fire(k, buf, sem):
        pltpu.async_copy(src_at(k), buf, sem)

    def scat(k, buf, sem):
        pltpu.make_async_copy(src_at(k), buf, sem).wait()
        pltpu.sync_copy(ones_v, deg_sh.at[buf], add=True)

    fire(0, idx_a, sem_a)
    fire(1, idx_b, sem_b)

    def triple(t, _):
        k = 3 * t
        fire(k + 2, idx_c, sem_c)
        scat(k, idx_a, sem_a)

        @pl.when(t + 1 < nk // 3)
        def _():
            fire(k + 3, idx_a, sem_a)

        scat(k + 1, idx_b, sem_b)

        @pl.when(t + 1 < nk // 3)
        def _():
            fire(k + 4, idx_b, sem_b)

        scat(k + 2, idx_c, sem_c)
        return 0

    lax.fori_loop(0, nk // 3, triple, 0, unroll=False)

    @pl.when(s == 0)
    def _():
        pltpu.sync_copy(dst_hbm.at[pl.ds(base + (DCHUNKS - 1) * CHUNK,
                                         CHUNK)], idx_a)
        pltpu.sync_copy(ones_v, deg_sh.at[idx_a], add=True)

    plsc.subcore_barrier()

    sl = pl.ds(s * ROWS_PER_TILE, ROWS_PER_TILE)
    pltpu.sync_copy(deg_sh.at[sl], deg_hbm.at[c, sl])


@functools.partial(
    pl.kernel,
    out_type=(
        jax.ShapeDtypeStruct((N, H), jnp.float32),
        jax.ShapeDtypeStruct((N, H), jnp.float32),
    ),
    mesh=_MESH,
    scratch_types=[
        pltpu.VMEM((2, CHUNK), jnp.int32),
        pltpu.VMEM((2, CHUNK), jnp.int32),
        pltpu.VMEM((2, CHUNK), jnp.int32),
        pltpu.VMEM((2, CHUNK), jnp.int32),
        pltpu.VMEM((CHUNK, H), jnp.float32),
        pltpu.VMEM((CHUNK, H), jnp.float32),
        pltpu.VMEM_SHARED((NPAD, H), jnp.float32),
        pltpu.SemaphoreType.DMA,
        pltpu.SemaphoreType.DMA,
        pltpu.SemaphoreType.DMA,
        pltpu.SemaphoreType.DMA,
        pltpu.SemaphoreType.DMA,
        pltpu.SemaphoreType.DMA,
    ],
)
def _edge_kernel(zlo_hbm, zhi_hbm, src_hbm, dst_hbm, outlo_hbm, outhi_hbm,
                 idx_a, idx_b, idx_a2, idx_b2, rows_v0, rows_v1, acc_sh,
                 sem_la, sem_lb, sem_la2, sem_lb2, sem_g0, sem_g1):
    c = lax.axis_index("c")
    s = lax.axis_index("s")

    last = N - 15 * ROWS_PER_TILE

    def init(z_ref):
        @pl.when(s < 15)
        def _():
            sl = pl.ds(s * ROWS_PER_TILE, ROWS_PER_TILE)
            pltpu.sync_copy(z_ref.at[sl], acc_sh.at[sl])

        @pl.when(s == 15)
        def _():
            sl = pl.ds(15 * ROWS_PER_TILE, last)
            pltpu.sync_copy(z_ref.at[sl], acc_sh.at[sl])

    nk = (ECHUNKS // 16)

    def edges(z_ref):
        def ld(k, buf, sem):
            b = (s + 16 * k) * CHUNK
            pltpu.async_copy(src_hbm.at[pl.ds(b, CHUNK)], buf.at[0], sem)
            pltpu.async_copy(dst_hbm.at[pl.ds(b, CHUNK)], buf.at[1], sem)

        def ld_wait(k, buf, sem):
            b = (s + 16 * k) * CHUNK
            pltpu.make_async_copy(src_hbm.at[pl.ds(b, CHUNK)], buf.at[0],
                                  sem).wait()
            pltpu.make_async_copy(dst_hbm.at[pl.ds(b, CHUNK)], buf.at[1],
                                  sem).wait()

        def g_fire(buf, rows, sem):
            pltpu.async_copy(z_ref.at[buf.at[0]], rows, sem)

        def g_wait(buf, rows, sem):
            pltpu.make_async_copy(z_ref.at[buf.at[0]], rows, sem).wait()

        def scat(buf, rows):
            pltpu.sync_copy(rows, acc_sh.at[buf.at[1]], add=True)

        ld(0, idx_a, sem_la)
        ld(1, idx_b, sem_lb)
        ld_wait(0, idx_a, sem_la)
        g_fire(idx_a, rows_v0, sem_g0)
        ld_wait(1, idx_b, sem_lb)
        g_fire(idx_b, rows_v1, sem_g1)

        def quad(q, _):
            k = 4 * q
            ld(k + 2, idx_a2, sem_la2)
            ld(k + 3, idx_b2, sem_lb2)
            g_wait(idx_a, rows_v0, sem_g0)
            scat(idx_a, rows_v0)
            ld_wait(k + 2, idx_a2, sem_la2)
            g_fire(idx_a2, rows_v0, sem_g0)
            g_wait(idx_b, rows_v1, sem_g1)
            scat(idx_b, rows_v1)
            ld_wait(k + 3, idx_b2, sem_lb2)
            g_fire(idx_b2, rows_v1, sem_g1)
            ld(k + 4, idx_a, sem_la)
            ld(k + 5, idx_b, sem_lb)
            g_wait(idx_a2, rows_v0, sem_g0)
            scat(idx_a2, rows_v0)
            ld_wait(k + 4, idx_a, sem_la)
            g_fire(idx_a, rows_v0, sem_g0)
            g_wait(idx_b2, rows_v1, sem_g1)
            scat(idx_b2, rows_v1)
            ld_wait(k + 5, idx_b, sem_lb)
            g_fire(idx_b, rows_v1, sem_g1)
            return 0

        lax.fori_loop(0, (nk - 2) // 4, quad, 0, unroll=False)

        g_wait(idx_a, rows_v0, sem_g0)
        scat(idx_a, rows_v0)
        g_wait(idx_b, rows_v1, sem_g1)
        scat(idx_b, rows_v1)

        @pl.when(s < ECHUNKS - 16 * nk)
        def _():
            b = (16 * nk + s) * CHUNK
            pltpu.sync_copy(src_hbm.at[pl.ds(b, CHUNK)], idx_a.at[0])
            pltpu.sync_copy(dst_hbm.at[pl.ds(b, CHUNK)], idx_a.at[1])
            pltpu.async_copy(z_ref.at[idx_a.at[0]], rows_v0, sem_g0).wait()
            pltpu.sync_copy(rows_v0, acc_sh.at[idx_a.at[1]], add=True)

    def writeout(out_ref):
        @pl.when(s < 15)
        def _():
            sl = pl.ds(s * ROWS_PER_TILE, ROWS_PER_TILE)
            pltpu.sync_copy(acc_sh.at[sl], out_ref.at[sl])

        @pl.when(s == 15)
        def _():
            sl = pl.ds(15 * ROWS_PER_TILE, last)
            pltpu.sync_copy(acc_sh.at[sl], out_ref.at[sl])

    @pl.when(c == 0)
    def _():
        init(zlo_hbm)

    @pl.when(c == 1)
    def _():
        init(zhi_hbm)

    plsc.subcore_barrier()

    @pl.when(c == 0)
    def _():
        edges(zlo_hbm)

    @pl.when(c == 1)
    def _():
        edges(zhi_hbm)

    plsc.subcore_barrier()

    @pl.when(c == 0)
    def _():
        writeout(outlo_hbm)

    @pl.when(c == 1)
    def _():
        writeout(outhi_hbm)


RBLK = 400


def _project_body(x_ref, x0_ref, w1_ref, w2_ref, deg0_ref, deg1_ref, zlo_ref,
                  zhi_ref, base_ref, dinv_ref):
    deg = deg0_ref[...] + deg1_ref[...] + 1.0
    dinv = lax.rsqrt(deg)
    y = lax.dot_general(x_ref[...], w1_ref[...], (((1,), (1,)), ((), ())),
                        preferred_element_type=jnp.float32,
                        precision=lax.Precision.HIGHEST)
    z = (0.9 * dinv) * y
    zlo_ref[...] = z[:, :H]
    zhi_ref[...] = z[:, H:]
    base_ref[...] = 0.1 * lax.dot_general(
        x0_ref[...], w2_ref[...], (((1,), (1,)), ((), ())),
        preferred_element_type=jnp.float32,
        precision=lax.Precision.HIGHEST)
    dinv_ref[...] = dinv


def _finish_body(acclo_ref, acchi_ref, base_ref, dinv_ref, out_ref):
    acc = jnp.concatenate([acclo_ref[...], acchi_ref[...]], axis=1)
    out_ref[...] = base_ref[...] + dinv_ref[...] * acc


def _project(x, x0, W1, W2, deg2d0, deg2d1):
    grid = (N // RBLK,)
    return pl.pallas_call(
        _project_body,
        grid=grid,
        in_specs=[
            pl.BlockSpec((RBLK, D), lambda i: (i, 0)),
            pl.BlockSpec((RBLK, D), lambda i: (i, 0)),
            pl.BlockSpec((D, D), lambda i: (0, 0)),
            pl.BlockSpec((D, D), lambda i: (0, 0)),
            pl.BlockSpec((RBLK, 1), lambda i: (i, 0)),
            pl.BlockSpec((RBLK, 1), lambda i: (i, 0)),
        ],
        out_specs=[
            pl.BlockSpec((RBLK, H), lambda i: (i, 0)),
            pl.BlockSpec((RBLK, H), lambda i: (i, 0)),
            pl.BlockSpec((RBLK, D), lambda i: (i, 0)),
            pl.BlockSpec((RBLK, 1), lambda i: (i, 0)),
        ],
        out_shape=[
            jax.ShapeDtypeStruct((N, H), jnp.float32),
            jax.ShapeDtypeStruct((N, H), jnp.float32),
            jax.ShapeDtypeStruct((N, D), jnp.float32),
            jax.ShapeDtypeStruct((N, 1), jnp.float32),
        ],
    )(x, x0, W1, W2, deg2d0, deg2d1)


def _finish(acc_lo, acc_hi, base, dinv2d):
    grid = (N // RBLK,)
    return pl.pallas_call(
        _finish_body,
        grid=grid,
        in_specs=[
            pl.BlockSpec((RBLK, H), lambda i: (i, 0)),
            pl.BlockSpec((RBLK, H), lambda i: (i, 0)),
            pl.BlockSpec((RBLK, D), lambda i: (i, 0)),
            pl.BlockSpec((RBLK, 1), lambda i: (i, 0)),
        ],
        out_specs=pl.BlockSpec((RBLK, D), lambda i: (i, 0)),
        out_shape=jax.ShapeDtypeStruct((N, D), jnp.float32),
    )(acc_lo, acc_hi, base, dinv2d)


def kernel(x, x0, edge_index, W1, W2):
    src = edge_index[0].astype(jnp.int32)
    dst = edge_index[1].astype(jnp.int32)

    degp = _degree_kernel(dst)
    deg2d0 = degp[0, :N].reshape(N, 1)
    deg2d1 = degp[1, :N].reshape(N, 1)

    z_lo, z_hi, base, dinv2d = _project(x, x0, W1, W2, deg2d0, deg2d1)
    acc_lo, acc_hi = _edge_kernel(z_lo, z_hi, src, dst)
    return _finish(acc_lo, acc_hi, base, dinv2d)


# speedup vs baseline: 22.6000x; 1.1072x over previous
import functools

import jax
import jax.numpy as jnp
from jax import lax
from jax.experimental import pallas as pl
from jax.experimental.pallas import tpu as pltpu
from jax.experimental.pallas import tpu_sc as plsc

N = 10000
E = 160000
D = 256
H = D // 2
NPAD = 10240
ROWS_PER_TILE = 640
CHUNK = 128
ECHUNKS = E // CHUNK
EPAIRS = (ECHUNKS // 16) // 2
DCHUNKS = (E // 2) // CHUNK

_MESH = plsc.VectorSubcoreMesh(core_axis_name="c", subcore_axis_name="s")


def _fill(ref, n, value):
    vec = jnp.full((16,), value, dtype=ref.dtype)

    def body(i, _):
        ref[pl.ds(i * 16, 16)] = vec
        return 0

    lax.fori_loop(0, n // 16, body, 0, unroll=False)


@functools.partial(
    pl.kernel,
    out_type=jax.ShapeDtypeStruct((2, NPAD), jnp.float32),
    mesh=_MESH,
    scratch_types=[
        pltpu.VMEM((CHUNK,), jnp.int32),
        pltpu.VMEM((CHUNK,), jnp.int32),
        pltpu.VMEM((CHUNK,), jnp.int32),
        pltpu.VMEM((CHUNK,), jnp.float32),
        pltpu.VMEM((ROWS_PER_TILE,), jnp.float32),
        pltpu.VMEM_SHARED((NPAD,), jnp.float32),
        pltpu.SemaphoreType.DMA,
        pltpu.SemaphoreType.DMA,
        pltpu.SemaphoreType.DMA,
    ],
)
def _degree_kernel(dst_hbm, deg_hbm, idx_a, idx_b, idx_c, ones_v, zeros_v,
                   deg_sh, sem_a, sem_b, sem_c):
    c = lax.axis_index("c")
    s = lax.axis_index("s")

    _fill(zeros_v, ROWS_PER_TILE, 0.0)
    _fill(ones_v, CHUNK, 1.0)
    pltpu.sync_copy(zeros_v, deg_sh.at[pl.ds(s * ROWS_PER_TILE,
                                             ROWS_PER_TILE)])

    plsc.subcore_barrier()

    base = c * (E // 2)
    nk = DCHUNKS // 16

    def src_at(k):
        return dst_hbm.at[pl.ds(base + (s + 16 * k) * CHUNK, CHUNK)]

    def fire(k, buf, sem):
        pltpu.async_copy(src_at(k), buf, sem)

    def scat(k, buf, sem):
        pltpu.make_async_copy(src_at(k), buf, sem).wait()
        pltpu.sync_copy(ones_v, deg_sh.at[buf], add=True)

    fire(0, idx_a, sem_a)
    fire(1, idx_b, sem_b)

    def triple(t, _):
        k = 3 * t
        fire(k + 2, idx_c, sem_c)
        scat(k, idx_a, sem_a)

        @pl.when(t + 1 < nk // 3)
        def _():
            fire(k + 3, idx_a, sem_a)

        scat(k + 1, idx_b, sem_b)

        @pl.when(t + 1 < nk // 3)
        def _():
            fire(k + 4, idx_b, sem_b)

        scat(k + 2, idx_c, sem_c)
        return 0

    lax.fori_loop(0, nk // 3, triple, 0, unroll=False)

    @pl.when(s == 0)
    def _():
        pltpu.sync_copy(dst_hbm.at[pl.ds(base + (DCHUNKS - 1) * CHUNK,
                                         CHUNK)], idx_a)
        pltpu.sync_copy(ones_v, deg_sh.at[idx_a], add=True)

    plsc.subcore_barrier()

    sl = pl.ds(s * ROWS_PER_TILE, ROWS_PER_TILE)
    pltpu.sync_copy(deg_sh.at[sl], deg_hbm.at[c, sl])


@functools.partial(
    pl.kernel,
    out_type=(
        jax.ShapeDtypeStruct((N, H), jnp.float32),
        jax.ShapeDtypeStruct((N, H), jnp.float32),
    ),
    mesh=_MESH,
    scratch_types=[
        pltpu.VMEM((2, CHUNK), jnp.int32),
        pltpu.VMEM((2, CHUNK), jnp.int32),
        pltpu.VMEM((2, CHUNK), jnp.int32),
        pltpu.VMEM((2, CHUNK), jnp.int32),
        pltpu.VMEM((CHUNK, H), jnp.float32),
        pltpu.VMEM((CHUNK, H), jnp.float32),
        pltpu.VMEM_SHARED((NPAD, H), jnp.float32),
        pltpu.SemaphoreType.DMA,
        pltpu.SemaphoreType.DMA,
        pltpu.SemaphoreType.DMA,
        pltpu.SemaphoreType.DMA,
        pltpu.SemaphoreType.DMA,
        pltpu.SemaphoreType.DMA,
    ],
)
def _edge_kernel(zlo_hbm, zhi_hbm, src_hbm, dst_hbm, outlo_hbm, outhi_hbm,
                 idx_a, idx_b, idx_a2, idx_b2, rows_v0, rows_v1, acc_sh,
                 sem_la, sem_lb, sem_la2, sem_lb2, sem_g0, sem_g1):
    c = lax.axis_index("c")
    s = lax.axis_index("s")

    last = N - 15 * ROWS_PER_TILE

    def init(z_ref):
        @pl.when(s < 15)
        def _():
            sl = pl.ds(s * ROWS_PER_TILE, ROWS_PER_TILE)
            pltpu.sync_copy(z_ref.at[sl], acc_sh.at[sl])

        @pl.when(s == 15)
        def _():
            sl = pl.ds(15 * ROWS_PER_TILE, last)
            pltpu.sync_copy(z_ref.at[sl], acc_sh.at[sl])

    nk = (ECHUNKS // 16)

    def edges(z_ref):
        def ld(k, buf, sem):
            b = (s + 16 * k) * CHUNK
            pltpu.async_copy(src_hbm.at[pl.ds(b, CHUNK)], buf.at[0], sem)
            pltpu.async_copy(dst_hbm.at[pl.ds(b, CHUNK)], buf.at[1], sem)

        def ld_wait(k, buf, sem):
            b = (s + 16 * k) * CHUNK
            pltpu.make_async_copy(src_hbm.at[pl.ds(b, CHUNK)], buf.at[0],
                                  sem).wait()
            pltpu.make_async_copy(dst_hbm.at[pl.ds(b, CHUNK)], buf.at[1],
                                  sem).wait()

        def g_fire(buf, rows, sem):
            pltpu.async_copy(z_ref.at[buf.at[0]], rows, sem)

        def g_wait(buf, rows, sem):
            pltpu.make_async_copy(z_ref.at[buf.at[0]], rows, sem).wait()

        def scat(buf, rows):
            pltpu.sync_copy(rows, acc_sh.at[buf.at[1]], add=True)

        ld(0, idx_a, sem_la)
        ld(1, idx_b, sem_lb)
        ld_wait(0, idx_a, sem_la)
        g_fire(idx_a, rows_v0, sem_g0)
        ld_wait(1, idx_b, sem_lb)
        g_fire(idx_b, rows_v1, sem_g1)

        def quad(q, _):
            k = 4 * q
            ld(k + 2, idx_a2, sem_la2)
            ld(k + 3, idx_b2, sem_lb2)
            g_wait(idx_a, rows_v0, sem_g0)
            scat(idx_a, rows_v0)
            ld_wait(k + 2, idx_a2, sem_la2)
            g_fire(idx_a2, rows_v0, sem_g0)
            g_wait(idx_b, rows_v1, sem_g1)
            scat(idx_b, rows_v1)
            ld_wait(k + 3, idx_b2, sem_lb2)
            g_fire(idx_b2, rows_v1, sem_g1)
            ld(k + 4, idx_a, sem_la)
            ld(k + 5, idx_b, sem_lb)
            g_wait(idx_a2, rows_v0, sem_g0)
            scat(idx_a2, rows_v0)
            ld_wait(k + 4, idx_a, sem_la)
            g_fire(idx_a, rows_v0, sem_g0)
            g_wait(idx_b2, rows_v1, sem_g1)
            scat(idx_b2, rows_v1)
            ld_wait(k + 5, idx_b, sem_lb)
            g_fire(idx_b, rows_v1, sem_g1)
            return 0

        lax.fori_loop(0, (nk - 2) // 4, quad, 0, unroll=False)

        g_wait(idx_a, rows_v0, sem_g0)
        scat(idx_a, rows_v0)
        g_wait(idx_b, rows_v1, sem_g1)
        scat(idx_b, rows_v1)

        @pl.when(s < ECHUNKS - 16 * nk)
        def _():
            b = (16 * nk + s) * CHUNK
            pltpu.sync_copy(src_hbm.at[pl.ds(b, CHUNK)], idx_a.at[0])
            pltpu.sync_copy(dst_hbm.at[pl.ds(b, CHUNK)], idx_a.at[1])
            pltpu.async_copy(z_ref.at[idx_a.at[0]], rows_v0, sem_g0).wait()
            pltpu.sync_copy(rows_v0, acc_sh.at[idx_a.at[1]], add=True)

    def writeout(out_ref):
        @pl.when(s < 15)
        def _():
            sl = pl.ds(s * ROWS_PER_TILE, ROWS_PER_TILE)
            pltpu.sync_copy(acc_sh.at[sl], out_ref.at[sl])

        @pl.when(s == 15)
        def _():
            sl = pl.ds(15 * ROWS_PER_TILE, last)
            pltpu.sync_copy(acc_sh.at[sl], out_ref.at[sl])

    @pl.when(c == 0)
    def _():
        init(zlo_hbm)

    @pl.when(c == 1)
    def _():
        init(zhi_hbm)

    plsc.subcore_barrier()

    @pl.when(c == 0)
    def _():
        edges(zlo_hbm)

    @pl.when(c == 1)
    def _():
        edges(zhi_hbm)

    plsc.subcore_barrier()

    @pl.when(c == 0)
    def _():
        writeout(outlo_hbm)

    @pl.when(c == 1)
    def _():
        writeout(outhi_hbm)


RBLK = 1000
_PREC = lax.Precision.HIGHEST


def _project_body(x_ref, w1_ref, deg0_ref, deg1_ref, zlo_ref, zhi_ref,
                  dinv_ref):
    deg = deg0_ref[...] + deg1_ref[...] + 1.0
    dinv = lax.rsqrt(deg)
    y = lax.dot_general(x_ref[...], w1_ref[...], (((1,), (1,)), ((), ())),
                        preferred_element_type=jnp.float32, precision=_PREC)
    z = (0.9 * dinv) * y
    zlo_ref[...] = z[:, :H]
    zhi_ref[...] = z[:, H:]
    dinv_ref[...] = dinv


def _finish_body(acclo_ref, acchi_ref, x0_ref, w2_ref, dinv_ref, out_ref):
    base = 0.1 * lax.dot_general(
        x0_ref[...], w2_ref[...], (((1,), (1,)), ((), ())),
        preferred_element_type=jnp.float32, precision=_PREC)
    acc = jnp.concatenate([acclo_ref[...], acchi_ref[...]], axis=1)
    out_ref[...] = base + dinv_ref[...] * acc


def _project(x, W1, deg2d0, deg2d1):
    grid = (N // RBLK,)
    return pl.pallas_call(
        _project_body,
        grid=grid,
        in_specs=[
            pl.BlockSpec((RBLK, D), lambda i: (i, 0)),
            pl.BlockSpec((D, D), lambda i: (0, 0)),
            pl.BlockSpec((RBLK, 1), lambda i: (i, 0)),
            pl.BlockSpec((RBLK, 1), lambda i: (i, 0)),
        ],
        out_specs=[
            pl.BlockSpec((RBLK, H), lambda i: (i, 0)),
            pl.BlockSpec((RBLK, H), lambda i: (i, 0)),
            pl.BlockSpec((RBLK, 1), lambda i: (i, 0)),
        ],
        out_shape=[
            jax.ShapeDtypeStruct((N, H), jnp.float32),
            jax.ShapeDtypeStruct((N, H), jnp.float32),
            jax.ShapeDtypeStruct((N, 1), jnp.float32),
        ],
    )(x, W1, deg2d0, deg2d1)


def _finish(acc_lo, acc_hi, x0, W2, dinv2d):
    grid = (N // RBLK,)
    return pl.pallas_call(
        _finish_body,
        grid=grid,
        in_specs=[
            pl.BlockSpec((RBLK, H), lambda i: (i, 0)),
            pl.BlockSpec((RBLK, H), lambda i: (i, 0)),
            pl.BlockSpec((RBLK, D), lambda i: (i, 0)),
            pl.BlockSpec((D, D), lambda i: (0, 0)),
            pl.BlockSpec((RBLK, 1), lambda i: (i, 0)),
        ],
        out_specs=pl.BlockSpec((RBLK, D), lambda i: (i, 0)),
        out_shape=jax.ShapeDtypeStruct((N, D), jnp.float32),
    )(acc_lo, acc_hi, x0, W2, dinv2d)


def kernel(x, x0, edge_index, W1, W2):
    src = edge_index[0].astype(jnp.int32)
    dst = edge_index[1].astype(jnp.int32)

    degp = _degree_kernel(dst)
    deg2d0 = degp[0, :N].reshape(N, 1)
    deg2d1 = degp[1, :N].reshape(N, 1)

    z_lo, z_hi, dinv2d = _project(x, W1, deg2d0, deg2d1)
    acc_lo, acc_hi = _edge_kernel(z_lo, z_hi, src, dst)
    return _finish(acc_lo, acc_hi, x0, W2, dinv2d)


# speedup vs baseline: 22.9992x; 1.0177x over previous
import functools

import jax
import jax.numpy as jnp
from jax import lax
from jax.experimental import pallas as pl
from jax.experimental.pallas import tpu as pltpu
from jax.experimental.pallas import tpu_sc as plsc

N = 10000
E = 160000
D = 256
H = D // 2
NPAD = 10240
ROWS_PER_TILE = 640
CHUNK = 128
ECHUNKS = E // CHUNK
EPAIRS = (ECHUNKS // 16) // 2
DCHUNKS = (E // 2) // CHUNK

_MESH = plsc.VectorSubcoreMesh(core_axis_name="c", subcore_axis_name="s")


def _fill(ref, n, value):
    vec = jnp.full((16,), value, dtype=ref.dtype)

    def body(i, _):
        ref[pl.ds(i * 16, 16)] = vec
        return 0

    lax.fori_loop(0, n // 16, body, 0, unroll=False)


@functools.partial(
    pl.kernel,
    out_type=jax.ShapeDtypeStruct((2, NPAD), jnp.float32),
    mesh=_MESH,
    scratch_types=[
        pltpu.VMEM((CHUNK,), jnp.int32),
        pltpu.VMEM((CHUNK,), jnp.int32),
        pltpu.VMEM((CHUNK,), jnp.int32),
        pltpu.VMEM((CHUNK,), jnp.float32),
        pltpu.VMEM((ROWS_PER_TILE,), jnp.float32),
        pltpu.VMEM_SHARED((NPAD,), jnp.float32),
        pltpu.SemaphoreType.DMA,
        pltpu.SemaphoreType.DMA,
        pltpu.SemaphoreType.DMA,
    ],
)
def _degree_kernel(dst_hbm, deg_hbm, idx_a, idx_b, idx_c, ones_v, zeros_v,
                   deg_sh, sem_a, sem_b, sem_c):
    c = lax.axis_index("c")
    s = lax.axis_index("s")

    _fill(zeros_v, ROWS_PER_TILE, 0.0)
    _fill(ones_v, CHUNK, 1.0)
    pltpu.sync_copy(zeros_v, deg_sh.at[pl.ds(s * ROWS_PER_TILE,
                                             ROWS_PER_TILE)])

    plsc.subcore_barrier()

    base = c * (E // 2)
    nk = DCHUNKS // 16

    def src_at(k):
        return dst_hbm.at[pl.ds(base + (s + 16 * k) * CHUNK, CHUNK)]

    def fire(k, buf, sem):
        pltpu.async_copy(src_at(k), buf, sem)

    def scat(k, buf, sem):
        pltpu.make_async_copy(src_at(k), buf, sem).wait()
        pltpu.sync_copy(ones_v, deg_sh.at[buf], add=True)

    fire(0, idx_a, sem_a)
    fire(1, idx_b, sem_b)

    def triple(t, _):
        k = 3 * t
        fire(k + 2, idx_c, sem_c)
        scat(k, idx_a, sem_a)

        @pl.when(t + 1 < nk // 3)
        def _():
            fire(k + 3, idx_a, sem_a)

        scat(k + 1, idx_b, sem_b)

        @pl.when(t + 1 < nk // 3)
        def _():
            fire(k + 4, idx_b, sem_b)

        scat(k + 2, idx_c, sem_c)
        return 0

    lax.fori_loop(0, nk // 3, triple, 0, unroll=False)

    @pl.when(s == 0)
    def _():
        pltpu.sync_copy(dst_hbm.at[pl.ds(base + (DCHUNKS - 1) * CHUNK,
                                         CHUNK)], idx_a)
        pltpu.sync_copy(ones_v, deg_sh.at[idx_a], add=True)

    plsc.subcore_barrier()

    sl = pl.ds(s * ROWS_PER_TILE, ROWS_PER_TILE)
    pltpu.sync_copy(deg_sh.at[sl], deg_hbm.at[c, sl])


@functools.partial(
    pl.kernel,
    out_type=(
        jax.ShapeDtypeStruct((N, H), jnp.float32),
        jax.ShapeDtypeStruct((N, H), jnp.float32),
    ),
    mesh=_MESH,
    scratch_types=[
        pltpu.VMEM((2, CHUNK), jnp.int32),
        pltpu.VMEM((2, CHUNK), jnp.int32),
        pltpu.VMEM((2, CHUNK), jnp.int32),
        pltpu.VMEM((2, CHUNK), jnp.int32),
        pltpu.VMEM((CHUNK, H), jnp.float32),
        pltpu.VMEM((CHUNK, H), jnp.float32),
        pltpu.VMEM_SHARED((NPAD, H), jnp.float32),
        pltpu.SemaphoreType.DMA,
        pltpu.SemaphoreType.DMA,
        pltpu.SemaphoreType.DMA,
        pltpu.SemaphoreType.DMA,
        pltpu.SemaphoreType.DMA,
        pltpu.SemaphoreType.DMA,
    ],
)
def _edge_kernel(zlo_hbm, zhi_hbm, src_hbm, dst_hbm, outlo_hbm, outhi_hbm,
                 idx_a, idx_b, idx_a2, idx_b2, rows_v0, rows_v1, acc_sh,
                 sem_la, sem_lb, sem_la2, sem_lb2, sem_g0, sem_g1):
    c = lax.axis_index("c")
    s = lax.axis_index("s")

    last = N - 15 * ROWS_PER_TILE

    def init(z_ref):
        @pl.when(s < 15)
        def _():
            sl = pl.ds(s * ROWS_PER_TILE, ROWS_PER_TILE)
            pltpu.sync_copy(z_ref.at[sl], acc_sh.at[sl])

        @pl.when(s == 15)
        def _():
            sl = pl.ds(15 * ROWS_PER_TILE, last)
            pltpu.sync_copy(z_ref.at[sl], acc_sh.at[sl])

    nk = (ECHUNKS // 16)

    def edges(z_ref):
        def ld(k, buf, sem):
            b = (s + 16 * k) * CHUNK
            pltpu.async_copy(src_hbm.at[pl.ds(b, CHUNK)], buf.at[0], sem)
            pltpu.async_copy(dst_hbm.at[pl.ds(b, CHUNK)], buf.at[1], sem)

        def ld_wait(k, buf, sem):
            b = (s + 16 * k) * CHUNK
            pltpu.make_async_copy(src_hbm.at[pl.ds(b, CHUNK)], buf.at[0],
                                  sem).wait()
            pltpu.make_async_copy(dst_hbm.at[pl.ds(b, CHUNK)], buf.at[1],
                                  sem).wait()

        def g_fire(buf, rows, sem):
            pltpu.async_copy(z_ref.at[buf.at[0]], rows, sem)

        def g_wait(buf, rows, sem):
            pltpu.make_async_copy(z_ref.at[buf.at[0]], rows, sem).wait()

        def scat(buf, rows):
            pltpu.sync_copy(rows, acc_sh.at[buf.at[1]], add=True)

        ld(0, idx_a, sem_la)
        ld(1, idx_b, sem_lb)
        ld_wait(0, idx_a, sem_la)
        g_fire(idx_a, rows_v0, sem_g0)
        ld_wait(1, idx_b, sem_lb)
        g_fire(idx_b, rows_v1, sem_g1)

        def quad(q, _):
            k = 4 * q
            ld(k + 2, idx_a2, sem_la2)
            ld(k + 3, idx_b2, sem_lb2)
            g_wait(idx_a, rows_v0, sem_g0)
            scat(idx_a, rows_v0)
            ld_wait(k + 2, idx_a2, sem_la2)
            g_fire(idx_a2, rows_v0, sem_g0)
            g_wait(idx_b, rows_v1, sem_g1)
            scat(idx_b, rows_v1)
            ld_wait(k + 3, idx_b2, sem_lb2)
            g_fire(idx_b2, rows_v1, sem_g1)
            ld(k + 4, idx_a, sem_la)
            ld(k + 5, idx_b, sem_lb)
            g_wait(idx_a2, rows_v0, sem_g0)
            scat(idx_a2, rows_v0)
            ld_wait(k + 4, idx_a, sem_la)
            g_fire(idx_a, rows_v0, sem_g0)
            g_wait(idx_b2, rows_v1, sem_g1)
            scat(idx_b2, rows_v1)
            ld_wait(k + 5, idx_b, sem_lb)
            g_fire(idx_b, rows_v1, sem_g1)
            return 0

        lax.fori_loop(0, (nk - 2) // 4, quad, 0, unroll=False)

        g_wait(idx_a, rows_v0, sem_g0)
        scat(idx_a, rows_v0)
        g_wait(idx_b, rows_v1, sem_g1)
        scat(idx_b, rows_v1)

        @pl.when(s < ECHUNKS - 16 * nk)
        def _():
            b = (16 * nk + s) * CHUNK
            pltpu.sync_copy(src_hbm.at[pl.ds(b, CHUNK)], idx_a.at[0])
            pltpu.sync_copy(dst_hbm.at[pl.ds(b, CHUNK)], idx_a.at[1])
            pltpu.async_copy(z_ref.at[idx_a.at[0]], rows_v0, sem_g0).wait()
            pltpu.sync_copy(rows_v0, acc_sh.at[idx_a.at[1]], add=True)

    def writeout(out_ref):
        @pl.when(s < 15)
        def _():
            sl = pl.ds(s * ROWS_PER_TILE, ROWS_PER_TILE)
            pltpu.sync_copy(acc_sh.at[sl], out_ref.at[sl])

        @pl.when(s == 15)
        def _():
            sl = pl.ds(15 * ROWS_PER_TILE, last)
            pltpu.sync_copy(acc_sh.at[sl], out_ref.at[sl])

    @pl.when(c == 0)
    def _():
        init(zlo_hbm)

    @pl.when(c == 1)
    def _():
        init(zhi_hbm)

    plsc.subcore_barrier()

    @pl.when(c == 0)
    def _():
        edges(zlo_hbm)

    @pl.when(c == 1)
    def _():
        edges(zhi_hbm)

    plsc.subcore_barrier()

    @pl.when(c == 0)
    def _():
        writeout(outlo_hbm)

    @pl.when(c == 1)
    def _():
        writeout(outhi_hbm)


RBLK = 1000
_PREC = lax.Precision.HIGHEST


def _project_body(x_ref, w1_ref, deg0_ref, deg1_ref, zlo_ref, zhi_ref,
                  dinv_ref):
    deg = deg0_ref[...] + deg1_ref[...] + 1.0
    dinv = lax.rsqrt(deg)
    y = lax.dot_general(x_ref[...], w1_ref[...], (((1,), (1,)), ((), ())),
                        preferred_element_type=jnp.float32, precision=_PREC)
    z = (0.9 * dinv) * y
    zlo_ref[...] = z[:, :H]
    zhi_ref[...] = z[:, H:]
    dinv_ref[...] = dinv


def _base_body(x0_ref, w2_ref, base_ref):
    base_ref[...] = 0.1 * lax.dot_general(
        x0_ref[...], w2_ref[...], (((1,), (1,)), ((), ())),
        preferred_element_type=jnp.float32, precision=_PREC)


def _finish_body(acclo_ref, acchi_ref, base_ref, dinv_ref, out_ref):
    acc = jnp.concatenate([acclo_ref[...], acchi_ref[...]], axis=1)
    out_ref[...] = base_ref[...] + dinv_ref[...] * acc


def _project(x, W1, deg2d0, deg2d1):
    grid = (N // RBLK,)
    return pl.pallas_call(
        _project_body,
        grid=grid,
        in_specs=[
            pl.BlockSpec((RBLK, D), lambda i: (i, 0)),
            pl.BlockSpec((D, D), lambda i: (0, 0)),
            pl.BlockSpec((RBLK, 1), lambda i: (i, 0)),
            pl.BlockSpec((RBLK, 1), lambda i: (i, 0)),
        ],
        out_specs=[
            pl.BlockSpec((RBLK, H), lambda i: (i, 0)),
            pl.BlockSpec((RBLK, H), lambda i: (i, 0)),
            pl.BlockSpec((RBLK, 1), lambda i: (i, 0)),
        ],
        out_shape=[
            jax.ShapeDtypeStruct((N, H), jnp.float32),
            jax.ShapeDtypeStruct((N, H), jnp.float32),
            jax.ShapeDtypeStruct((N, 1), jnp.float32),
        ],
    )(x, W1, deg2d0, deg2d1)


def _base(x0, W2):
    grid = (N // RBLK,)
    return pl.pallas_call(
        _base_body,
        grid=grid,
        in_specs=[
            pl.BlockSpec((RBLK, D), lambda i: (i, 0)),
            pl.BlockSpec((D, D), lambda i: (0, 0)),
        ],
        out_specs=pl.BlockSpec((RBLK, D), lambda i: (i, 0)),
        out_shape=jax.ShapeDtypeStruct((N, D), jnp.float32),
    )(x0, W2)


def _finish(acc_lo, acc_hi, base, dinv2d):
    grid = (N // RBLK,)
    return pl.pallas_call(
        _finish_body,
        grid=grid,
        in_specs=[
            pl.BlockSpec((RBLK, H), lambda i: (i, 0)),
            pl.BlockSpec((RBLK, H), lambda i: (i, 0)),
            pl.BlockSpec((RBLK, D), lambda i: (i, 0)),
            pl.BlockSpec((RBLK, 1), lambda i: (i, 0)),
        ],
        out_specs=pl.BlockSpec((RBLK, D), lambda i: (i, 0)),
        out_shape=jax.ShapeDtypeStruct((N, D), jnp.float32),
    )(acc_lo, acc_hi, base, dinv2d)


def kernel(x, x0, edge_index, W1, W2):
    src = edge_index[0].astype(jnp.int32)
    dst = edge_index[1].astype(jnp.int32)

    degp = _degree_kernel(dst)
    deg2d0 = degp[0, :N].reshape(N, 1)
    deg2d1 = degp[1, :N].reshape(N, 1)

    z_lo, z_hi, dinv2d = _project(x, W1, deg2d0, deg2d1)
    base = _base(x0, W2)
    acc_lo, acc_hi = _edge_kernel(z_lo, z_hi, src, dst)
    return _finish(acc_lo, acc_hi, base, dinv2d)
